# baseline (speedup 1.0000x reference)
import functools

import jax
import jax.numpy as jnp
from jax import lax
from jax.experimental import pallas as pl
from jax.experimental.pallas import tpu as pltpu
from jax.experimental.pallas import tpu_sc as plsc

IMG_W = 512.0
IMG_H = 512.0

_NC = 2
_NS = 16
_NW = _NC * _NS

_L_PER_W = 1024 // _NW
_GROUPS = _L_PER_W // 16
_C = 256
_OUT_PER_W = _L_PER_W * _C


def _splat(vec, i):
    idx = jnp.full((16, 1), i, jnp.int32)
    dnums = lax.GatherDimensionNumbers(
        offset_dims=(), collapsed_slice_dims=(0,), start_index_map=(0,))
    return lax.gather(vec, idx, dnums, (1,),
                      mode=lax.GatherScatterMode.PROMISE_IN_BOUNDS)


def _sc_body(rows_hbm, bbt_hbm, wt_hbm, b_hbm, out_hbm,
             bb_v, wt_v, b_v, idx_v, feat_v, out_v, sem, sem2):
    cell_w = jnp.float32(IMG_W / 32.0)
    cell_h = jnp.float32(IMG_H / 32.0)

    wid = lax.axis_index("s") * _NC + lax.axis_index("c")

    wt_cp = pltpu.async_copy(wt_hbm, wt_v, sem2)
    b_cp = pltpu.async_copy(b_hbm, b_v, sem2)
    pltpu.sync_copy(bbt_hbm.at[wid], bb_v)

    iota = lax.iota(jnp.int32, 16)

    lab_g = []
    for g in range(_GROUPS):
        x = bb_v[0, pl.ds(g * 16, 16)]
        y = bb_v[1, pl.ds(g * 16, 16)]
        w = bb_v[2, pl.ds(g * 16, 16)]
        h = bb_v[3, pl.ds(g * 16, 16)]
        xc = x + (w / 2.0).astype(jnp.int32).astype(jnp.float32)
        yc = y + (h / 2.0).astype(jnp.int32).astype(jnp.float32)
        cx = (xc / cell_w).astype(jnp.int32)
        cy = (yc / cell_h).astype(jnp.int32)
        lid = wid * _L_PER_W + g * 16 + iota
        bi = lax.shift_right_logical(lid, 3)
        r0 = ((bi * 32 + cy) * 64
              + lax.shift_right_logical(cx, 3) * 16 + (cx & 7))
        idx_v[pl.ds(g * 16, 16)] = r0
        idx_v[pl.ds(32 + g * 16, 16)] = r0 + 8

        cxf = cx.astype(jnp.float32)
        cyf = cy.astype(jnp.float32)
        lab_g.append(((xc - cxf * cell_w) / cell_w,
                      (yc - cyf * cell_h) / cell_h,
                      w / IMG_W,
                      h / IMG_H))

    copies = []
    for s in range(4):
        copies.append(pltpu.async_copy(
            rows_hbm.at[idx_v.at[pl.ds(s * 16, 16)]],
            feat_v.at[pl.ds(s * 16, 16)], sem))
    wt_cp.wait()
    b_cp.wait()
    for cp in copies:
        cp.wait()

    for half in range(2):
        wr = [[wt_v[k, pl.ds(half * 128 + j * 16, 16)] for j in range(8)]
              for k in range(4)]
        br = [b_v[0, pl.ds(half * 128 + j * 16, 16)] for j in range(8)]
        for g in range(_GROUPS):
            lx, ly, lw, lh = lab_g[g]

            def _fuse(lq, _, half=half, g=g, wr=wr, br=br,
                      lx=lx, ly=ly, lw=lw, lh=lh):
                for u in range(2):
                    li = lq * 2 + u
                    l = g * 16 + li
                    lxb = _splat(lx, li)
                    lyb = _splat(ly, li)
                    lwb = _splat(lw, li)
                    lhb = _splat(lh, li)
                    frow = half * _L_PER_W + l
                    orow = (lax.shift_right_logical(l, 3) * 16
                            + half * 8 + (l & 7))
                    for j in range(8):
                        acc = (feat_v[frow, pl.ds(j * 16, 16)]
                               + lxb * wr[0][j] + lyb * wr[1][j]
                               + lwb * wr[2][j] + lhb * wr[3][j] + br[j])
                        out_v[pl.ds(orow * 128 + j * 16, 16)] = acc
                return 0

            lax.fori_loop(0, 8, _fuse, 0)

    pltpu.sync_copy(out_v, out_hbm.at[pl.ds(wid * _OUT_PER_W, _OUT_PER_W)])


def kernel(input, bboxes, W, b):
    B, K, N, _ = bboxes.shape
    C = input.shape[1]
    fh, fw = input.shape[2], input.shape[3]
    rows = (input.transpose(0, 2, 3, 1)
            .reshape(B * K, fh, fw // 8, 8, C // 128, 128)
            .transpose(0, 1, 2, 4, 3, 5)
            .reshape(B * K * fh * (fw // 8) * (C // 128) * 8, 128))
    bbt = (bboxes.reshape(_NW, _L_PER_W, 4)
           .transpose(0, 2, 1).reshape(_NW, 4, _L_PER_W))
    wt = W.T
    b2 = b.reshape(1, C)

    mesh = plsc.VectorSubcoreMesh(core_axis_name="c", subcore_axis_name="s")
    run = functools.partial(
        pl.kernel, mesh=mesh,
        out_type=jax.ShapeDtypeStruct((B * K * N * C,), jnp.float32),
        scratch_types=[
            pltpu.VMEM((4, _L_PER_W), jnp.float32),
            pltpu.VMEM((4, _C), jnp.float32),
            pltpu.VMEM((1, _C), jnp.float32),
            pltpu.VMEM((2 * _L_PER_W,), jnp.int32),
            pltpu.VMEM((2 * _L_PER_W, 128), jnp.float32),
            pltpu.VMEM((_OUT_PER_W,), jnp.float32),
            pltpu.SemaphoreType.DMA,
            pltpu.SemaphoreType.DMA,
        ],
    )(_sc_body)
    out = run(rows, bbt, wt, b2)
    return (out.reshape(B * K, C // 128, N, 128)
            .transpose(0, 2, 1, 3)
            .reshape(B, K, N, C))


# speedup vs baseline: 1.0491x; 1.0491x over previous
import functools

import jax
import jax.numpy as jnp
from jax import lax
from jax.experimental import pallas as pl
from jax.experimental.pallas import tpu as pltpu
from jax.experimental.pallas import tpu_sc as plsc

IMG_W = 512.0
IMG_H = 512.0

_NC = 2
_NS = 16
_NW = _NC * _NS

_L_PER_W = 1024 // _NW
_GROUPS = _L_PER_W // 16
_C = 256
_OUT_PER_W = _L_PER_W * _C


def _splat(vec, i):
    idx = jnp.full((16, 1), i, jnp.int32)
    dnums = lax.GatherDimensionNumbers(
        offset_dims=(), collapsed_slice_dims=(0,), start_index_map=(0,))
    return lax.gather(vec, idx, dnums, (1,),
                      mode=lax.GatherScatterMode.PROMISE_IN_BOUNDS)


def _sc_body(rows_hbm, bbt_hbm, wt_hbm, b_hbm, out_hbm,
             bb_v, wt_v, b_v, idx_v, feat_v, out_v,
             sem0, sem1, sem2, sem3, semw):
    cell_w = jnp.float32(IMG_W / 32.0)
    cell_h = jnp.float32(IMG_H / 32.0)

    wid = lax.axis_index("s") * _NC + lax.axis_index("c")

    wt_cp = pltpu.async_copy(wt_hbm, wt_v, semw)
    b_cp = pltpu.async_copy(b_hbm, b_v, semw)
    pltpu.sync_copy(bbt_hbm.at[wid], bb_v)

    iota = lax.iota(jnp.int32, 16)

    lab_g = []
    for g in range(_GROUPS):
        x = bb_v[0, pl.ds(g * 16, 16)]
        y = bb_v[1, pl.ds(g * 16, 16)]
        w = bb_v[2, pl.ds(g * 16, 16)]
        h = bb_v[3, pl.ds(g * 16, 16)]
        xc = x + (w / 2.0).astype(jnp.int32).astype(jnp.float32)
        yc = y + (h / 2.0).astype(jnp.int32).astype(jnp.float32)
        cx = (xc / cell_w).astype(jnp.int32)
        cy = (yc / cell_h).astype(jnp.int32)
        lid = wid * _L_PER_W + g * 16 + iota
        bi = lax.shift_right_logical(lid, 3)
        r0 = ((bi * 32 + cy) * 64
              + lax.shift_right_logical(cx, 3) * 16 + (cx & 7))
        idx_v[pl.ds(g * 16, 16)] = r0
        idx_v[pl.ds(32 + g * 16, 16)] = r0 + 8

        cxf = cx.astype(jnp.float32)
        cyf = cy.astype(jnp.float32)
        lab_g.append(((xc - cxf * cell_w) / cell_w,
                      (yc - cyf * cell_h) / cell_h,
                      w / IMG_W,
                      h / IMG_H))

    sems = [sem0, sem1, sem2, sem3]
    copies = []
    for s in range(4):
        copies.append(pltpu.async_copy(
            rows_hbm.at[idx_v.at[pl.ds(((s % 2) * 16 + (s // 2) * 32), 16)]],
            feat_v.at[pl.ds(((s % 2) * 16 + (s // 2) * 32), 16)], sems[s]))
    wt_cp.wait()
    b_cp.wait()

    for s in range(4):
        half, g = s // 2, s % 2
        copies[s].wait()
        wr = [[wt_v[k, pl.ds(half * 128 + j * 16, 16)] for j in range(8)]
              for k in range(4)]
        br = [b_v[0, pl.ds(half * 128 + j * 16, 16)] for j in range(8)]
        lx, ly, lw, lh = lab_g[g]

        def _fuse(li, _, half=half, g=g, wr=wr, br=br,
                  lx=lx, ly=ly, lw=lw, lh=lh):
            l = g * 16 + li
            lxb = _splat(lx, li)
            lyb = _splat(ly, li)
            lwb = _splat(lw, li)
            lhb = _splat(lh, li)
            frow = half * _L_PER_W + l
            orow = (lax.shift_right_logical(l, 3) * 16
                    + half * 8 + (l & 7))
            for j in range(8):
                acc = (feat_v[frow, pl.ds(j * 16, 16)]
                       + lxb * wr[0][j] + lyb * wr[1][j]
                       + lwb * wr[2][j] + lhb * wr[3][j] + br[j])
                out_v[pl.ds(orow * 128 + j * 16, 16)] = acc
            return 0

        lax.fori_loop(0, 16, _fuse, 0)

    pltpu.sync_copy(out_v, out_hbm.at[pl.ds(wid * _OUT_PER_W, _OUT_PER_W)])


def kernel(input, bboxes, W, b):
    B, K, N, _ = bboxes.shape
    C = input.shape[1]
    fh, fw = input.shape[2], input.shape[3]
    rows = (input.transpose(0, 2, 3, 1)
            .reshape(B * K, fh, fw // 8, 8, C // 128, 128)
            .transpose(0, 1, 2, 4, 3, 5)
            .reshape(B * K * fh * (fw // 8) * (C // 128) * 8, 128))
    bbt = (bboxes.reshape(_NW, _L_PER_W, 4)
           .transpose(0, 2, 1).reshape(_NW, 4, _L_PER_W))
    wt = W.T
    b2 = b.reshape(1, C)

    mesh = plsc.VectorSubcoreMesh(core_axis_name="c", subcore_axis_name="s")
    run = functools.partial(
        pl.kernel, mesh=mesh,
        out_type=jax.ShapeDtypeStruct((B * K * N * C,), jnp.float32),
        scratch_types=[
            pltpu.VMEM((4, _L_PER_W), jnp.float32),
            pltpu.VMEM((4, _C), jnp.float32),
            pltpu.VMEM((1, _C), jnp.float32),
            pltpu.VMEM((2 * _L_PER_W,), jnp.int32),
            pltpu.VMEM((2 * _L_PER_W, 128), jnp.float32),
            pltpu.VMEM((_OUT_PER_W,), jnp.float32),
            pltpu.SemaphoreType.DMA,
            pltpu.SemaphoreType.DMA,
            pltpu.SemaphoreType.DMA,
            pltpu.SemaphoreType.DMA,
            pltpu.SemaphoreType.DMA,
        ],
    )(_sc_body)
    out = run(rows, bbt, wt, b2)
    return (out.reshape(B * K, C // 128, N, 128)
            .transpose(0, 2, 1, 3)
            .reshape(B, K, N, C))
